# baseline (speedup 1.0000x reference)
import functools

import jax
import jax.numpy as jnp
from jax import lax
from jax.experimental import pallas as pl
from jax.experimental.pallas import tpu as pltpu
from jax.experimental.pallas import tpu_sc as plsc

_VOCAB = 100000
_EMB = 64
_SEQ = 8192
_BATCH = 4
_TOT = _BATCH * _SEQ
_NC = 2
_NS = 16
_NW = _NC * _NS
_PER_W = _TOT // _NW
_CHUNK = 128
_NCH = _PER_W // _CHUNK
_LANES = 16
_NB = 3
_DEPTH = 2


def _emb_body(xg_hbm, xo_hbm, tok2_hbm, pos2_hbm, out_hbm,
              idx_v, off_v, pos_v, rows_v, stage_v, gsem, osem, psem):
    cid = lax.axis_index("c")
    sid = lax.axis_index("s")
    wid = sid * _NC + cid
    base = pl.multiple_of(wid * _PER_W, _PER_W)
    pos_base = pl.multiple_of(lax.rem(base, _SEQ), _PER_W)

    xrow = pl.multiple_of(wid * _NCH, _NCH)
    pltpu.sync_copy(xg_hbm.at[pl.ds(xrow, _NCH)], idx_v)
    pltpu.sync_copy(xo_hbm.at[pl.ds(xrow, _NCH)], off_v)

    gathers = {}
    pos_cps = {}
    outs = {}
    for j in range(-_DEPTH, _NCH):
        f = j + _DEPTH
        if 0 <= f < _NCH:
            if f - _NB >= 0:
                outs[f - _NB].wait()
            gathers[f] = pltpu.async_copy(
                tok2_hbm.at[idx_v.at[f]], rows_v.at[f % _NB], gsem)
            pos_cps[f] = pltpu.async_copy(
                pos2_hbm.at[pl.ds(pl.multiple_of((pos_base + f * _CHUNK) // 2, _CHUNK // 2), _CHUNK // 2)],
                pos_v.at[f % _NB], psem)
        if j < 0:
            continue

        gathers[j].wait()
        pos_cps[j].wait()
        slot = j % _NB

        def half_add(r, _):
            offs = off_v[j, pl.ds(2 * r, _LANES)]
            for g in range(2 * _EMB // _LANES):
                sub = g // (_EMB // _LANES)
                rr = 2 * r + sub
                col = (g % (_EMB // _LANES)) * _LANES
                hoff = offs[sub]
                stage_v[slot, rr, pl.ds(col, _LANES)] = (
                    rows_v[slot, rr, pl.ds(hoff + col, _LANES)]
                    + pos_v[slot, r, pl.ds(sub * _EMB + col, _LANES)])
            return 0

        lax.fori_loop(0, _CHUNK // 2, half_add, 0, unroll=2)

        outs[j] = pltpu.async_copy(
            stage_v.at[slot],
            out_hbm.at[pl.ds(pl.multiple_of(base + j * _CHUNK, _CHUNK), _CHUNK)], osem)

    for j in range(_NCH - _NB, _NCH):
        if j >= 0:
            outs[j].wait()


@jax.jit
def _emb(xg, xo, tok2, pos2):
    mesh = plsc.VectorSubcoreMesh(core_axis_name="c", subcore_axis_name="s")
    run = functools.partial(
        pl.kernel,
        mesh=mesh,
        out_type=jax.ShapeDtypeStruct((_TOT, _EMB), jnp.float32),
        scratch_types=[
            pltpu.VMEM((_NCH, _CHUNK), jnp.int32),
            pltpu.VMEM((_NCH, _CHUNK), jnp.int32),
            pltpu.VMEM((_NB, _CHUNK // 2, 2 * _EMB), jnp.float32),
            pltpu.VMEM((_NB, _CHUNK, 2 * _EMB), jnp.float32),
            pltpu.VMEM((_NB, _CHUNK, _EMB), jnp.float32),
            pltpu.SemaphoreType.DMA,
            pltpu.SemaphoreType.DMA,
            pltpu.SemaphoreType.DMA,
        ],
        compiler_params=pltpu.CompilerParams(use_tc_tiling_on_sc=True),
    )(_emb_body)
    return run(xg, xo, tok2, pos2)


def kernel(x, token_table, pos_table):
    xi = x.astype(jnp.int32).reshape(_NW * _NCH, _CHUNK)
    xg = xi >> 1
    xo = (xi & 1) * _EMB
    tok2 = token_table.reshape(_VOCAB // 2, 2 * _EMB)
    pos2 = pos_table.reshape(_SEQ // 2, 2 * _EMB)
    out = _emb(xg, xo, tok2, pos2)
    return out.reshape(_BATCH, _SEQ, _EMB)


# speedup vs baseline: 1.0545x; 1.0545x over previous
import functools

import jax
import jax.numpy as jnp
from jax import lax
from jax.experimental import pallas as pl
from jax.experimental.pallas import tpu as pltpu
from jax.experimental.pallas import tpu_sc as plsc

_VOCAB = 100000
_EMB = 64
_SEQ = 8192
_BATCH = 4
_TOT = _BATCH * _SEQ
_NC = 2
_NS = 16
_NW = _NC * _NS
_PER_W = _TOT // _NW
_CHUNK = 128
_NCH = _PER_W // _CHUNK
_LANES = 16
_NB = 3
_DEPTH = 2


def _emb_body(xg_hbm, xo_hbm, tok2_hbm, pos_hbm, out_hbm,
              idx_v, off_v, rows_v, stage_v, gsem, osem, psem):
    cid = lax.axis_index("c")
    sid = lax.axis_index("s")
    wid = sid * _NC + cid
    base = pl.multiple_of(wid * _PER_W, _PER_W)
    pos_base = pl.multiple_of(lax.rem(base, _SEQ), _PER_W)

    xrow = pl.multiple_of(wid * _NCH, _NCH)
    pltpu.sync_copy(xg_hbm.at[pl.ds(xrow, _NCH)], idx_v)
    pltpu.sync_copy(xo_hbm.at[pl.ds(xrow, _NCH)], off_v)

    gathers = {}
    pos_cps = {}
    outs = {}
    for j in range(-_DEPTH, _NCH):
        f = j + _DEPTH
        if 0 <= f < _NCH:
            if f - _NB >= 0:
                outs[f - _NB].wait()
            gathers[f] = pltpu.async_copy(
                tok2_hbm.at[idx_v.at[f]], rows_v.at[f % _NB], gsem)
            pos_cps[f] = pltpu.async_copy(
                pos_hbm.at[pl.ds(pl.multiple_of(pos_base + f * _CHUNK, _CHUNK), _CHUNK)],
                stage_v.at[f % _NB], psem)
        if j < 0:
            continue

        gathers[j].wait()
        pos_cps[j].wait()
        slot = j % _NB

        def half_add(r, _):
            offs = off_v[j, pl.ds(2 * r, _LANES)]
            for g in range(2 * _EMB // _LANES):
                sub = g // (_EMB // _LANES)
                rr = 2 * r + sub
                col = (g % (_EMB // _LANES)) * _LANES
                hoff = offs[sub]
                plsc.addupdate(
                    stage_v.at[slot, rr, pl.ds(col, _LANES)],
                    rows_v[slot, rr, pl.ds(hoff + col, _LANES)])
            return 0

        lax.fori_loop(0, _CHUNK // 2, half_add, 0, unroll=2)

        outs[j] = pltpu.async_copy(
            stage_v.at[slot],
            out_hbm.at[pl.ds(pl.multiple_of(base + j * _CHUNK, _CHUNK), _CHUNK)], osem)

    for j in range(_NCH - _NB, _NCH):
        if j >= 0:
            outs[j].wait()


@jax.jit
def _emb(xg, xo, tok2, pos_table):
    mesh = plsc.VectorSubcoreMesh(core_axis_name="c", subcore_axis_name="s")
    run = functools.partial(
        pl.kernel,
        mesh=mesh,
        out_type=jax.ShapeDtypeStruct((_TOT, _EMB), jnp.float32),
        scratch_types=[
            pltpu.VMEM((_NCH, _CHUNK), jnp.int32),
            pltpu.VMEM((_NCH, _CHUNK), jnp.int32),
            pltpu.VMEM((_NB, _CHUNK, 2 * _EMB), jnp.float32),
            pltpu.VMEM((_NB, _CHUNK, _EMB), jnp.float32),
            pltpu.SemaphoreType.DMA,
            pltpu.SemaphoreType.DMA,
            pltpu.SemaphoreType.DMA,
        ],
        compiler_params=pltpu.CompilerParams(use_tc_tiling_on_sc=True),
    )(_emb_body)
    return run(xg, xo, tok2, pos_table)


def kernel(x, token_table, pos_table):
    xi = x.astype(jnp.int32).reshape(_NW * _NCH, _CHUNK)
    xg = xi >> 1
    xo = (xi & 1) * _EMB
    tok2 = token_table.reshape(_VOCAB // 2, 2 * _EMB)
    out = _emb(xg, xo, tok2, pos_table)
    return out.reshape(_BATCH, _SEQ, _EMB)
